# baseline (speedup 1.0000x reference)
import functools

import jax
import jax.numpy as jnp
import numpy as np
from jax import lax
from jax.experimental import pallas as pl
from jax.experimental.pallas import tpu as pltpu
from jax.experimental.pallas import tpu_sc as plsc

_NUM_TILES = 32
_LANES = 16
_LANE_TILE = 128
_CH = 256

_ROT0 = (13, 15, 26, 6)
_ROT1 = (17, 29, 16, 24)


def _tf2x32(k1, k2, x1, x2):
    ks = [np.uint32(k1), np.uint32(k2),
          np.uint32(k1 ^ k2 ^ np.uint32(0x1BD11BDA))]
    x = [np.asarray(x1, np.uint32) + ks[0], np.asarray(x2, np.uint32) + ks[1]]

    def rounds(x, rots):
        for r in rots:
            x0 = x[0] + x[1]
            x1r = (x[1] << np.uint32(r)) | (x[1] >> np.uint32(32 - r))
            x = [x0, x0 ^ x1r]
        return x

    seq = ((_ROT0, 1, 2, 1), (_ROT1, 2, 0, 2), (_ROT0, 0, 1, 3),
           (_ROT1, 1, 2, 4), (_ROT0, 2, 0, 5))
    for rots, i0, i1, c in seq:
        x = rounds(x, rots)
        x = [x[0] + ks[i0], x[1] + ks[i1] + np.uint32(c)]
    return x[0], x[1]


def _split_key(k):
    b1, b2 = _tf2x32(k[0], k[1], np.zeros(2, np.uint32),
                     np.arange(2, dtype=np.uint32))
    return (b1[0], b2[0]), (b1[1], b2[1])


def _random_bits(k, n):
    b1, b2 = _tf2x32(k[0], k[1], np.zeros(n, np.uint32),
                     np.arange(n, dtype=np.uint32))
    return b1 ^ b2


def _randint(k, n, minval, maxval):
    k1, k2 = _split_key(k)
    hi, lo = _random_bits(k1, n), _random_bits(k2, n)
    span = np.uint32(maxval - minval)
    mult = np.uint32(np.uint32(2 ** 16) % span)
    mult = np.uint32((mult * mult) % span)
    off = ((hi % span) * mult + (lo % span)) % span
    return (np.int32(minval) + off.astype(np.int32)).astype(np.int32)


@functools.lru_cache(maxsize=None)
def _index_constants(b, l):
    max_window = l // 2
    key = (np.uint32(0), np.uint32(1))
    k1, k2 = _split_key(key)
    idx = _randint(k1, b, 0, l - max_window)
    offset = _randint(k2, b, 1, max_window)
    return idx, (idx + offset).astype(np.int32)


@functools.lru_cache(maxsize=None)
def _build_sc(B, D, L):
    assert B == _NUM_TILES and D % _CH == 0
    n_chunks = D // _CH

    mesh = plsc.VectorSubcoreMesh(core_axis_name="c", subcore_axis_name="s")

    def body(x_hbm, cols_hbm, out_hbm, cols_v, buf0, buf1, obuf, sem):
        b = lax.axis_index("s") * 2 + lax.axis_index("c")

        pltpu.sync_copy(cols_hbm, cols_v)
        col = cols_v[pl.ds(b, _LANES)][0]
        win = (col // _LANE_TILE) * _LANE_TILE
        lane = col - win

        bufs = (buf0, buf1)
        iota = lax.iota(jnp.int32, _LANES)
        lane_v = jnp.broadcast_to(lane, (_LANES,))

        def start(h):
            return pltpu.async_copy(
                x_hbm.at[b, pl.ds(h * _CH, _CH), pl.ds(win, _LANE_TILE)],
                bufs[h % 2], sem)

        pending = start(0)
        for h in range(n_chunks):
            nxt = start(h + 1) if h + 1 < n_chunks else None
            pending.wait()
            buf = bufs[h % 2]
            for j in range(_CH // _LANES):
                rows = j * _LANES + iota
                v = plsc.load_gather(buf, [rows, lane_v])
                obuf[pl.ds(h * _CH + j * _LANES, _LANES)] = v
            if nxt is not None:
                pending = nxt

        pltpu.sync_copy(obuf, out_hbm.at[b])

    return pl.kernel(
        body,
        out_type=jax.ShapeDtypeStruct((B, D), jnp.float32),
        mesh=mesh,
        compiler_params=pltpu.CompilerParams(needs_layout_passes=False),
        scratch_types=[
            pltpu.VMEM((B + _LANES,), jnp.int32),
            pltpu.VMEM((_CH, _LANE_TILE), jnp.float32),
            pltpu.VMEM((_CH, _LANE_TILE), jnp.float32),
            pltpu.VMEM((D,), jnp.float32),
            pltpu.SemaphoreType.DMA,
        ],
    )


@functools.lru_cache(maxsize=None)
def _build_tc(B, D, L):
    def body(winlane_ref, x_ref, out_ref):
        b = pl.program_id(0)
        lane = winlane_ref[1, b]
        sel = lax.broadcasted_iota(jnp.int32, (1, D, _LANE_TILE), 2) == lane
        r = jnp.sum(jnp.where(sel, x_ref[...], 0.0), axis=2)
        out_ref[...] = r[:, None, :]

    grid_spec = pltpu.PrefetchScalarGridSpec(
        num_scalar_prefetch=1,
        grid=(B,),
        in_specs=[
            pl.BlockSpec((1, D, _LANE_TILE),
                         lambda b, winlane: (b, 0, winlane[0, b])),
        ],
        out_specs=pl.BlockSpec((1, 1, D), lambda b, winlane: (b, 0, 0)),
    )
    return pl.pallas_call(
        body,
        grid_spec=grid_spec,
        out_shape=jax.ShapeDtypeStruct((B, 1, D), jnp.float32),
    )


def kernel(x_a, x_b):
    b, d, l = x_a.shape
    cols_a, cols_b = _index_constants(b, l)
    winlane_a = np.stack([cols_a // _LANE_TILE, cols_a % _LANE_TILE])
    cols_b_pad = np.concatenate(
        [cols_b, np.zeros((_LANES,), np.int32)]).astype(np.int32)

    out_b = _build_sc(b, d, l)(x_b, jnp.asarray(cols_b_pad))
    out_a = _build_tc(b, d, l)(jnp.asarray(winlane_a.astype(np.int32)), x_a)
    return (out_a.reshape(b, d), out_b)


# speedup vs baseline: 1.1159x; 1.1159x over previous
import functools

import jax
import jax.numpy as jnp
import numpy as np
from jax import lax
from jax.experimental import pallas as pl
from jax.experimental.pallas import tpu as pltpu
from jax.experimental.pallas import tpu_sc as plsc

_NUM_TILES = 32
_LANES = 16
_LANE_TILE = 128
_CH = 256

_ROT0 = (13, 15, 26, 6)
_ROT1 = (17, 29, 16, 24)


def _tf2x32(k1, k2, x1, x2):
    ks = [np.uint32(k1), np.uint32(k2),
          np.uint32(k1 ^ k2 ^ np.uint32(0x1BD11BDA))]
    x = [np.asarray(x1, np.uint32) + ks[0], np.asarray(x2, np.uint32) + ks[1]]

    def rounds(x, rots):
        for r in rots:
            x0 = x[0] + x[1]
            x1r = (x[1] << np.uint32(r)) | (x[1] >> np.uint32(32 - r))
            x = [x0, x0 ^ x1r]
        return x

    seq = ((_ROT0, 1, 2, 1), (_ROT1, 2, 0, 2), (_ROT0, 0, 1, 3),
           (_ROT1, 1, 2, 4), (_ROT0, 2, 0, 5))
    for rots, i0, i1, c in seq:
        x = rounds(x, rots)
        x = [x[0] + ks[i0], x[1] + ks[i1] + np.uint32(c)]
    return x[0], x[1]


def _split_key(k):
    b1, b2 = _tf2x32(k[0], k[1], np.zeros(2, np.uint32),
                     np.arange(2, dtype=np.uint32))
    return (b1[0], b2[0]), (b1[1], b2[1])


def _random_bits(k, n):
    b1, b2 = _tf2x32(k[0], k[1], np.zeros(n, np.uint32),
                     np.arange(n, dtype=np.uint32))
    return b1 ^ b2


def _randint(k, n, minval, maxval):
    k1, k2 = _split_key(k)
    hi, lo = _random_bits(k1, n), _random_bits(k2, n)
    span = np.uint32(maxval - minval)
    mult = np.uint32(np.uint32(2 ** 16) % span)
    mult = np.uint32((mult * mult) % span)
    off = ((hi % span) * mult + (lo % span)) % span
    return (np.int32(minval) + off.astype(np.int32)).astype(np.int32)


@functools.lru_cache(maxsize=None)
def _index_constants(b, l):
    max_window = l // 2
    key = (np.uint32(0), np.uint32(1))
    k1, k2 = _split_key(key)
    idx = _randint(k1, b, 0, l - max_window)
    offset = _randint(k2, b, 1, max_window)
    return idx, (idx + offset).astype(np.int32)


@functools.lru_cache(maxsize=None)
def _build_sc(B, D, L):
    assert B == _NUM_TILES and D % _CH == 0
    n_chunks = D // _CH

    mesh = plsc.VectorSubcoreMesh(core_axis_name="c", subcore_axis_name="s")

    def body(x_hbm, cols_hbm, out_hbm, cols_v, buf0, buf1, obuf, sem):
        b = lax.axis_index("s") * 2 + lax.axis_index("c")

        pltpu.sync_copy(cols_hbm, cols_v)
        col = cols_v[pl.ds(b, _LANES)][0]
        win = (col // _LANE_TILE) * _LANE_TILE
        lane = col - win

        bufs = (buf0, buf1)
        iota = lax.iota(jnp.int32, _LANES)
        lane_v = jnp.broadcast_to(lane, (_LANES,))

        def start(h):
            return pltpu.async_copy(
                x_hbm.at[b, pl.ds(h * _CH, _CH), pl.ds(win, _LANE_TILE)],
                bufs[h % 2], sem)

        pending = start(0)
        for h in range(n_chunks):
            nxt = start(h + 1) if h + 1 < n_chunks else None
            pending.wait()
            buf = bufs[h % 2]
            for j in range(_CH // _LANES):
                rows = j * _LANES + iota
                v = plsc.load_gather(buf, [rows, lane_v])
                obuf[pl.ds(h * _CH + j * _LANES, _LANES)] = v
            if nxt is not None:
                pending = nxt

        pltpu.sync_copy(obuf, out_hbm.at[b])

    return pl.kernel(
        body,
        out_type=jax.ShapeDtypeStruct((B, D), jnp.float32),
        mesh=mesh,
        compiler_params=pltpu.CompilerParams(needs_layout_passes=False),
        scratch_types=[
            pltpu.VMEM((B + _LANES,), jnp.int32),
            pltpu.VMEM((_CH, _LANE_TILE), jnp.float32),
            pltpu.VMEM((_CH, _LANE_TILE), jnp.float32),
            pltpu.VMEM((D,), jnp.float32),
            pltpu.SemaphoreType.DMA,
        ],
    )


@functools.lru_cache(maxsize=None)
def _build_tc(B, D, L):
    def body(winlane_ref, x_ref, out_ref):
        b = pl.program_id(0)
        lane = winlane_ref[1, b]
        sel = lax.broadcasted_iota(jnp.int32, (1, D, _LANE_TILE), 2) == lane
        r = jnp.sum(jnp.where(sel, x_ref[...], 0.0), axis=2, keepdims=True)[0]
        cols = lax.broadcasted_iota(jnp.int32, (D, B), 1)
        out_ref[...] = jnp.where(cols == b, r, out_ref[...])

    grid_spec = pltpu.PrefetchScalarGridSpec(
        num_scalar_prefetch=1,
        grid=(B,),
        in_specs=[
            pl.BlockSpec((1, D, _LANE_TILE),
                         lambda b, winlane: (b, 0, winlane[0, b])),
        ],
        out_specs=pl.BlockSpec((D, B), lambda b, winlane: (0, 0)),
    )
    return pl.pallas_call(
        body,
        grid_spec=grid_spec,
        out_shape=jax.ShapeDtypeStruct((D, B), jnp.float32),
    )


def kernel(x_a, x_b):
    b, d, l = x_a.shape
    cols_a, cols_b = _index_constants(b, l)
    winlane_a = np.stack([cols_a // _LANE_TILE, cols_a % _LANE_TILE])
    cols_b_pad = np.concatenate(
        [cols_b, np.zeros((_LANES,), np.int32)]).astype(np.int32)

    out_b = _build_sc(b, d, l)(x_b, jnp.asarray(cols_b_pad))
    out_a = _build_tc(b, d, l)(jnp.asarray(winlane_a.astype(np.int32)), x_a)
    return (out_a.T, out_b)


# speedup vs baseline: 1.1687x; 1.0473x over previous
import functools

import jax
import jax.numpy as jnp
import numpy as np
from jax import lax
from jax.experimental import pallas as pl
from jax.experimental.pallas import tpu as pltpu
from jax.experimental.pallas import tpu_sc as plsc

_NUM_TILES = 32
_LANES = 16
_LANE_TILE = 128
_CH = 256

_ROT0 = (13, 15, 26, 6)
_ROT1 = (17, 29, 16, 24)


def _tf2x32(k1, k2, x1, x2):
    ks = [np.uint32(k1), np.uint32(k2),
          np.uint32(k1 ^ k2 ^ np.uint32(0x1BD11BDA))]
    x = [np.asarray(x1, np.uint32) + ks[0], np.asarray(x2, np.uint32) + ks[1]]

    def rounds(x, rots):
        for r in rots:
            x0 = x[0] + x[1]
            x1r = (x[1] << np.uint32(r)) | (x[1] >> np.uint32(32 - r))
            x = [x0, x0 ^ x1r]
        return x

    seq = ((_ROT0, 1, 2, 1), (_ROT1, 2, 0, 2), (_ROT0, 0, 1, 3),
           (_ROT1, 1, 2, 4), (_ROT0, 2, 0, 5))
    for rots, i0, i1, c in seq:
        x = rounds(x, rots)
        x = [x[0] + ks[i0], x[1] + ks[i1] + np.uint32(c)]
    return x[0], x[1]


def _split_key(k):
    b1, b2 = _tf2x32(k[0], k[1], np.zeros(2, np.uint32),
                     np.arange(2, dtype=np.uint32))
    return (b1[0], b2[0]), (b1[1], b2[1])


def _random_bits(k, n):
    b1, b2 = _tf2x32(k[0], k[1], np.zeros(n, np.uint32),
                     np.arange(n, dtype=np.uint32))
    return b1 ^ b2


def _randint(k, n, minval, maxval):
    k1, k2 = _split_key(k)
    hi, lo = _random_bits(k1, n), _random_bits(k2, n)
    span = np.uint32(maxval - minval)
    mult = np.uint32(np.uint32(2 ** 16) % span)
    mult = np.uint32((mult * mult) % span)
    off = ((hi % span) * mult + (lo % span)) % span
    return (np.int32(minval) + off.astype(np.int32)).astype(np.int32)


@functools.lru_cache(maxsize=None)
def _index_constants(b, l):
    max_window = l // 2
    key = (np.uint32(0), np.uint32(1))
    k1, k2 = _split_key(key)
    idx = _randint(k1, b, 0, l - max_window)
    offset = _randint(k2, b, 1, max_window)
    return idx, (idx + offset).astype(np.int32)


@functools.lru_cache(maxsize=None)
def _build_sc(B, D, L):
    assert B == _NUM_TILES and D % _CH == 0
    n_chunks = D // _CH

    mesh = plsc.VectorSubcoreMesh(core_axis_name="c", subcore_axis_name="s")

    def body(x_hbm, cols_hbm, out_hbm, cols_v, buf0, buf1, obuf, sem):
        b = lax.axis_index("s") * 2 + lax.axis_index("c")

        pltpu.sync_copy(cols_hbm, cols_v)
        col = cols_v[pl.ds(b, _LANES)][0]
        win = (col // _LANE_TILE) * _LANE_TILE
        lane = col - win

        bufs = (buf0, buf1)
        iota = lax.iota(jnp.int32, _LANES)
        lane_v = jnp.broadcast_to(lane, (_LANES,))

        def start(h):
            return pltpu.async_copy(
                x_hbm.at[b, pl.ds(h * _CH, _CH), pl.ds(win, _LANE_TILE)],
                bufs[h % 2], sem)

        pending = start(0)
        for h in range(n_chunks):
            nxt = start(h + 1) if h + 1 < n_chunks else None
            pending.wait()
            buf = bufs[h % 2]
            for j in range(_CH // _LANES):
                rows = j * _LANES + iota
                v = plsc.load_gather(buf, [rows, lane_v])
                obuf[pl.ds(h * _CH + j * _LANES, _LANES)] = v
            if nxt is not None:
                pending = nxt

        pltpu.sync_copy(obuf, out_hbm.at[b])

    return pl.kernel(
        body,
        out_type=jax.ShapeDtypeStruct((B, D), jnp.float32),
        mesh=mesh,
        compiler_params=pltpu.CompilerParams(needs_layout_passes=False),
        scratch_types=[
            pltpu.VMEM((B + _LANES,), jnp.int32),
            pltpu.VMEM((_CH, _LANE_TILE), jnp.float32),
            pltpu.VMEM((_CH, _LANE_TILE), jnp.float32),
            pltpu.VMEM((D,), jnp.float32),
            pltpu.SemaphoreType.DMA,
        ],
    )


@functools.lru_cache(maxsize=None)
def _build_tc(B, D, L):
    def body(winlane_ref, x_ref, out_ref):
        b = pl.program_id(0)
        lane = winlane_ref[1, b]
        sel = lax.broadcasted_iota(jnp.int32, (1, D, _LANE_TILE), 2) == lane
        r = jnp.sum(jnp.where(sel, x_ref[...], 0.0), axis=2, keepdims=True)[0]
        cols = lax.broadcasted_iota(jnp.int32, (D, B), 1)
        out_ref[...] = jnp.where(cols == b, r, out_ref[...])

    grid_spec = pltpu.PrefetchScalarGridSpec(
        num_scalar_prefetch=1,
        grid=(B,),
        in_specs=[
            pl.BlockSpec((1, D, _LANE_TILE),
                         lambda b, winlane: (b, 0, winlane[0, b])),
        ],
        out_specs=pl.BlockSpec((D, B), lambda b, winlane: (0, 0)),
    )
    return pl.pallas_call(
        body,
        grid_spec=grid_spec,
        out_shape=jax.ShapeDtypeStruct((D, B), jnp.float32),
    )


def kernel(x_a, x_b):
    b, d, l = x_a.shape
    cols_a, cols_b = _index_constants(b, l)
    pad = np.zeros((_LANES,), np.int32)
    cols_a_pad = np.concatenate([cols_a, pad]).astype(np.int32)
    cols_b_pad = np.concatenate([cols_b, pad]).astype(np.int32)

    sc = _build_sc(b, d, l)
    out_a = sc(x_a, jnp.asarray(cols_a_pad))
    out_b = sc(x_b, jnp.asarray(cols_b_pad))
    return (out_a, out_b)


# speedup vs baseline: 1.4096x; 1.2061x over previous
import functools

import jax
import jax.numpy as jnp
import numpy as np
from jax import lax
from jax.experimental import pallas as pl
from jax.experimental.pallas import tpu as pltpu
from jax.experimental.pallas import tpu_sc as plsc

_NUM_TILES = 32
_LANES = 16
_LANE_TILE = 128
_CH = 256

_ROT0 = (13, 15, 26, 6)
_ROT1 = (17, 29, 16, 24)


def _tf2x32(k1, k2, x1, x2):
    ks = [np.uint32(k1), np.uint32(k2),
          np.uint32(k1 ^ k2 ^ np.uint32(0x1BD11BDA))]
    x = [np.asarray(x1, np.uint32) + ks[0], np.asarray(x2, np.uint32) + ks[1]]

    def rounds(x, rots):
        for r in rots:
            x0 = x[0] + x[1]
            x1r = (x[1] << np.uint32(r)) | (x[1] >> np.uint32(32 - r))
            x = [x0, x0 ^ x1r]
        return x

    seq = ((_ROT0, 1, 2, 1), (_ROT1, 2, 0, 2), (_ROT0, 0, 1, 3),
           (_ROT1, 1, 2, 4), (_ROT0, 2, 0, 5))
    for rots, i0, i1, c in seq:
        x = rounds(x, rots)
        x = [x[0] + ks[i0], x[1] + ks[i1] + np.uint32(c)]
    return x[0], x[1]


def _split_key(k):
    b1, b2 = _tf2x32(k[0], k[1], np.zeros(2, np.uint32),
                     np.arange(2, dtype=np.uint32))
    return (b1[0], b2[0]), (b1[1], b2[1])


def _random_bits(k, n):
    b1, b2 = _tf2x32(k[0], k[1], np.zeros(n, np.uint32),
                     np.arange(n, dtype=np.uint32))
    return b1 ^ b2


def _randint(k, n, minval, maxval):
    k1, k2 = _split_key(k)
    hi, lo = _random_bits(k1, n), _random_bits(k2, n)
    span = np.uint32(maxval - minval)
    mult = np.uint32(np.uint32(2 ** 16) % span)
    mult = np.uint32((mult * mult) % span)
    off = ((hi % span) * mult + (lo % span)) % span
    return (np.int32(minval) + off.astype(np.int32)).astype(np.int32)


@functools.lru_cache(maxsize=None)
def _index_constants(b, l):
    max_window = l // 2
    key = (np.uint32(0), np.uint32(1))
    k1, k2 = _split_key(key)
    idx = _randint(k1, b, 0, l - max_window)
    offset = _randint(k2, b, 1, max_window)
    return idx, (idx + offset).astype(np.int32)


@functools.lru_cache(maxsize=None)
def _build_sc(B, D, L):
    assert B == _NUM_TILES and D % _CH == 0
    n_chunks = D // _CH

    mesh = plsc.VectorSubcoreMesh(core_axis_name="c", subcore_axis_name="s")

    def body(x_hbm, cols_hbm, out_hbm, cols_v, buf0, buf1, obuf, sem):
        b = lax.axis_index("s") * 2 + lax.axis_index("c")

        pltpu.sync_copy(cols_hbm, cols_v)
        col = cols_v[pl.ds(b, _LANES)][0]
        win = (col // _LANE_TILE) * _LANE_TILE
        lane = col - win

        bufs = (buf0, buf1)
        iota = lax.iota(jnp.int32, _LANES)
        lane_v = jnp.broadcast_to(lane, (_LANES,))

        def start(h):
            return pltpu.async_copy(
                x_hbm.at[b, pl.ds(h * _CH, _CH), pl.ds(win, _LANE_TILE)],
                bufs[h % 2], sem)

        pending = start(0)
        for h in range(n_chunks):
            nxt = start(h + 1) if h + 1 < n_chunks else None
            pending.wait()
            buf = bufs[h % 2]
            for j in range(_CH // _LANES):
                rows = j * _LANES + iota
                v = plsc.load_gather(buf, [rows, lane_v])
                obuf[pl.ds(h * _CH + j * _LANES, _LANES)] = v
            if nxt is not None:
                pending = nxt

        pltpu.sync_copy(obuf, out_hbm.at[b])

    return pl.kernel(
        body,
        out_type=jax.ShapeDtypeStruct((B, D), jnp.float32),
        mesh=mesh,
        compiler_params=pltpu.CompilerParams(needs_layout_passes=False),
        scratch_types=[
            pltpu.VMEM((B + _LANES,), jnp.int32),
            pltpu.VMEM((_CH, _LANE_TILE), jnp.float32),
            pltpu.VMEM((_CH, _LANE_TILE), jnp.float32),
            pltpu.VMEM((D,), jnp.float32),
            pltpu.SemaphoreType.DMA,
        ],
    )


@functools.lru_cache(maxsize=None)
def _build_sc_fused(B, D, L):
    assert B == _NUM_TILES and D % _CH == 0
    n_chunks = D // _CH

    mesh = plsc.VectorSubcoreMesh(core_axis_name="c", subcore_axis_name="s")

    def body(xa_hbm, xb_hbm, cols_hbm, outa_hbm, outb_hbm,
             cols_v, buf0, buf1, obuf_a, obuf_b, sem):
        b = lax.axis_index("s") * 2 + lax.axis_index("c")

        pltpu.sync_copy(cols_hbm, cols_v)
        iota = lax.iota(jnp.int32, _LANES)
        bufs = (buf0, buf1)

        specs = []
        for base, x_hbm, obuf in ((0, xa_hbm, obuf_a), (B, xb_hbm, obuf_b)):
            col = cols_v[pl.ds(base + b, _LANES)][0]
            win = (col // _LANE_TILE) * _LANE_TILE
            lane_v = jnp.broadcast_to(col - win, (_LANES,))
            specs.append((x_hbm, win, lane_v, obuf))

        total = 2 * n_chunks

        def start(i):
            x_hbm, win, _, _ = specs[i // n_chunks]
            h = i % n_chunks
            return pltpu.async_copy(
                x_hbm.at[b, pl.ds(h * _CH, _CH), pl.ds(win, _LANE_TILE)],
                bufs[i % 2], sem)

        pending = start(0)
        for i in range(total):
            nxt = start(i + 1) if i + 1 < total else None
            pending.wait()
            buf = bufs[i % 2]
            _, _, lane_v, obuf = specs[i // n_chunks]
            h = i % n_chunks
            for j in range(_CH // _LANES):
                rows = j * _LANES + iota
                v = plsc.load_gather(buf, [rows, lane_v])
                obuf[pl.ds(h * _CH + j * _LANES, _LANES)] = v
            if nxt is not None:
                pending = nxt

        pltpu.sync_copy(obuf_a, outa_hbm.at[b])
        pltpu.sync_copy(obuf_b, outb_hbm.at[b])

    return pl.kernel(
        body,
        out_type=[jax.ShapeDtypeStruct((B, D), jnp.float32),
                  jax.ShapeDtypeStruct((B, D), jnp.float32)],
        mesh=mesh,
        compiler_params=pltpu.CompilerParams(needs_layout_passes=False),
        scratch_types=[
            pltpu.VMEM((2 * B + _LANES,), jnp.int32),
            pltpu.VMEM((_CH, _LANE_TILE), jnp.float32),
            pltpu.VMEM((_CH, _LANE_TILE), jnp.float32),
            pltpu.VMEM((D,), jnp.float32),
            pltpu.VMEM((D,), jnp.float32),
            pltpu.SemaphoreType.DMA,
        ],
    )


def kernel(x_a, x_b):
    b, d, l = x_a.shape
    cols_a, cols_b = _index_constants(b, l)
    pad = np.zeros((_LANES,), np.int32)
    cols = np.concatenate([cols_a, cols_b, pad]).astype(np.int32)

    out_a, out_b = _build_sc_fused(b, d, l)(x_a, x_b, jnp.asarray(cols))
    return (out_a, out_b)


# speedup vs baseline: 1.4135x; 1.0028x over previous
import functools

import jax
import jax.numpy as jnp
import numpy as np
from jax import lax
from jax.experimental import pallas as pl
from jax.experimental.pallas import tpu as pltpu
from jax.experimental.pallas import tpu_sc as plsc

_NUM_TILES = 32
_LANES = 16
_LANE_TILE = 128
_CH = 256

_ROT0 = (13, 15, 26, 6)
_ROT1 = (17, 29, 16, 24)


def _tf2x32(k1, k2, x1, x2):
    ks = [np.uint32(k1), np.uint32(k2),
          np.uint32(k1 ^ k2 ^ np.uint32(0x1BD11BDA))]
    x = [np.asarray(x1, np.uint32) + ks[0], np.asarray(x2, np.uint32) + ks[1]]

    def rounds(x, rots):
        for r in rots:
            x0 = x[0] + x[1]
            x1r = (x[1] << np.uint32(r)) | (x[1] >> np.uint32(32 - r))
            x = [x0, x0 ^ x1r]
        return x

    seq = ((_ROT0, 1, 2, 1), (_ROT1, 2, 0, 2), (_ROT0, 0, 1, 3),
           (_ROT1, 1, 2, 4), (_ROT0, 2, 0, 5))
    for rots, i0, i1, c in seq:
        x = rounds(x, rots)
        x = [x[0] + ks[i0], x[1] + ks[i1] + np.uint32(c)]
    return x[0], x[1]


def _split_key(k):
    b1, b2 = _tf2x32(k[0], k[1], np.zeros(2, np.uint32),
                     np.arange(2, dtype=np.uint32))
    return (b1[0], b2[0]), (b1[1], b2[1])


def _random_bits(k, n):
    b1, b2 = _tf2x32(k[0], k[1], np.zeros(n, np.uint32),
                     np.arange(n, dtype=np.uint32))
    return b1 ^ b2


def _randint(k, n, minval, maxval):
    k1, k2 = _split_key(k)
    hi, lo = _random_bits(k1, n), _random_bits(k2, n)
    span = np.uint32(maxval - minval)
    mult = np.uint32(np.uint32(2 ** 16) % span)
    mult = np.uint32((mult * mult) % span)
    off = ((hi % span) * mult + (lo % span)) % span
    return (np.int32(minval) + off.astype(np.int32)).astype(np.int32)


@functools.lru_cache(maxsize=None)
def _index_constants(b, l):
    max_window = l // 2
    key = (np.uint32(0), np.uint32(1))
    k1, k2 = _split_key(key)
    idx = _randint(k1, b, 0, l - max_window)
    offset = _randint(k2, b, 1, max_window)
    return idx, (idx + offset).astype(np.int32)


@functools.lru_cache(maxsize=None)
def _build_sc_fused(B, D, L):
    assert B == _NUM_TILES and D % _CH == 0
    n_chunks = D // _CH

    mesh = plsc.VectorSubcoreMesh(core_axis_name="c", subcore_axis_name="s")

    def body(xa_hbm, xb_hbm, cols_hbm, outa_hbm, outb_hbm,
             cols_v, buf0, buf1, obuf_a, obuf_b, sem):
        b = lax.axis_index("s") * 2 + lax.axis_index("c")

        pltpu.sync_copy(cols_hbm, cols_v)
        iota = lax.iota(jnp.int32, _LANES)
        bufs = (buf0, buf1)

        specs = []
        for base, x_hbm, obuf in ((0, xa_hbm, obuf_a), (B, xb_hbm, obuf_b)):
            col = cols_v[pl.ds(base + b, _LANES)][0]
            win = (col // _LANE_TILE) * _LANE_TILE
            lane_v = jnp.broadcast_to(col - win, (_LANES,))
            specs.append((x_hbm, win, lane_v, obuf))

        total = 2 * n_chunks

        def start(i):
            x_hbm, win, _, _ = specs[i // n_chunks]
            h = i % n_chunks
            return pltpu.async_copy(
                x_hbm.at[b, pl.ds(h * _CH, _CH), pl.ds(win, _LANE_TILE)],
                bufs[i % 2], sem)

        pending = start(0)
        for i in range(total):
            nxt = start(i + 1) if i + 1 < total else None
            pending.wait()
            buf = bufs[i % 2]
            _, _, lane_v, obuf = specs[i // n_chunks]
            h = i % n_chunks
            for j in range(_CH // _LANES):
                rows = j * _LANES + iota
                v = plsc.load_gather(buf, [rows, lane_v])
                obuf[pl.ds(h * _CH + j * _LANES, _LANES)] = v
            if nxt is not None:
                pending = nxt

        pltpu.sync_copy(obuf_a, outa_hbm.at[b])
        pltpu.sync_copy(obuf_b, outb_hbm.at[b])

    return pl.kernel(
        body,
        out_type=[jax.ShapeDtypeStruct((B, D), jnp.float32),
                  jax.ShapeDtypeStruct((B, D), jnp.float32)],
        mesh=mesh,
        compiler_params=pltpu.CompilerParams(needs_layout_passes=False),
        scratch_types=[
            pltpu.VMEM((2 * B + _LANES,), jnp.int32),
            pltpu.VMEM((_CH, _LANE_TILE), jnp.float32),
            pltpu.VMEM((_CH, _LANE_TILE), jnp.float32),
            pltpu.VMEM((D,), jnp.float32),
            pltpu.VMEM((D,), jnp.float32),
            pltpu.SemaphoreType.DMA,
        ],
    )


def kernel(x_a, x_b):
    b, d, l = x_a.shape
    cols_a, cols_b = _index_constants(b, l)
    pad = np.zeros((_LANES,), np.int32)
    cols = np.concatenate([cols_a, cols_b, pad]).astype(np.int32)

    out_a, out_b = _build_sc_fused(b, d, l)(x_a, x_b, jnp.asarray(cols))
    return (out_a, out_b)
